# baseline (speedup 1.0000x reference)
import functools

import jax
import jax.numpy as jnp
from jax.experimental import pallas as pl
from jax.experimental.pallas import tpu as pltpu

B, C, HW, P = 16, 3, 384, 16
HP = HW // P
T = HP * HP
FEAT, HID = 384, 64
K, NS = 512, 100
MOM, TS = 0.99, 0.07

N_TOK = B * T
CPP = C * P * P
RA = T
GA = B
NQ = K * NS
RB = 2048
GB = NQ // RB

_NEG_BIG = -3.0e38


def _norm_rows(x):
    n = jnp.sqrt(jnp.sum(x * x, axis=-1, keepdims=True))
    return x / jnp.clip(n, 1e-12)


def _patchify(x):
    x = x.reshape(B, C, HP, P, HP, P).transpose(0, 2, 4, 1, 3, 5)
    return x.reshape(B * T, CPP)


def _kernel_a(tok1_ref, tok2_ref,
              wp_ref, w1_ref, w2_ref, ws_ref,
              w1e_ref, w2e_ref, wse_ref, cent_ref,
              nz1_ref, z1_ref, z2_ref, idx_ref, gap_ref, mse_ref):
    i = pl.program_id(0)
    f32 = jnp.float32
    t1 = tok1_ref[...]
    t2 = tok2_ref[...]

    x1 = jnp.dot(t1, wp_ref[...], preferred_element_type=f32)
    h1 = jnp.dot(jnp.maximum(jnp.dot(x1, w1_ref[...], preferred_element_type=f32), 0.0),
                 w2_ref[...], preferred_element_type=f32)
    h1 = h1 + jnp.dot(x1, ws_ref[...], preferred_element_type=f32)
    z1_ref[0] = h1.T
    nz1 = _norm_rows(h1)
    nz1_ref[0] = nz1.T

    w1n = MOM * w1e_ref[...] + (1.0 - MOM) * w1_ref[...]
    w2n = MOM * w2e_ref[...] + (1.0 - MOM) * w2_ref[...]
    wsn = MOM * wse_ref[...] + (1.0 - MOM) * ws_ref[...]
    x2 = jnp.dot(t2, wp_ref[...].astype(jnp.bfloat16), preferred_element_type=f32)
    h2 = jnp.dot(jnp.maximum(jnp.dot(x2, w1n, preferred_element_type=f32), 0.0),
                 w2n, preferred_element_type=f32)
    h2 = h2 + jnp.dot(x2, wsn, preferred_element_type=f32)
    z2_ref[0] = h2.T
    nz2 = _norm_rows(h2)

    d = nz1 - nz2
    mse_part = jnp.sum(d * d) * (1.0 / (N_TOK * HID))

    cn = _norm_rows(cent_ref[...])
    cn2 = jnp.sum(cn * cn, axis=1)
    rn2 = jnp.sum(nz1 * nz1, axis=1, keepdims=True)
    s = jax.lax.dot_general(nz1, cn, (((1,), (1,)), ((), ())),
                            preferred_element_type=f32)
    neg = 2.0 * s - rn2 - cn2[None, :]
    m1 = jnp.max(neg, axis=1, keepdims=True)
    col = jax.lax.broadcasted_iota(jnp.int32, (RA, K), 1)
    idxv = jnp.min(jnp.where(neg == m1, col, K), axis=1)
    neg2 = jnp.where(col == idxv[:, None], _NEG_BIG, neg)
    m2 = jnp.max(neg2, axis=1)
    idx_ref[0, 0, :] = idxv
    gap_ref[0, 0, :] = m1[:, 0] - m2

    @pl.when(i == 0)
    def _():
        mse_ref[...] = mse_part.reshape(1, 1)

    @pl.when(i > 0)
    def _():
        mse_ref[...] += mse_part.reshape(1, 1)


def _kernel_b(q_ref, cent_ref, nce_ref):
    i = pl.program_id(0)
    f32 = jnp.float32

    qn = _norm_rows(q_ref[...])
    cn = _norm_rows(cent_ref[...])
    logits = jax.lax.dot_general(qn, cn, (((1,), (1,)), ((), ())),
                                 preferred_element_type=f32) * (1.0 / TS)
    lse = jnp.log(jnp.sum(jnp.exp(logits), axis=1))
    rows = i * RB + jax.lax.broadcasted_iota(jnp.int32, (RB, 1), 0)
    col = jax.lax.broadcasted_iota(jnp.int32, (RB, K), 1)
    hit = (rows >= NS * col) & (rows < NS * (col + 1))
    lab_logit = jnp.sum(jnp.where(hit, logits, 0.0), axis=1)
    part = jnp.sum(lse - lab_logit) * (1.0 / NQ)

    @pl.when(i == 0)
    def _():
        nce_ref[...] = part.reshape(1, 1)

    @pl.when(i > 0)
    def _():
        nce_ref[...] += part.reshape(1, 1)


@functools.partial(jax.jit)
def kernel(img, aug_img, Wp, W1, W2, Ws, W1e, W2e, Wse, centroid, queue):
    full = lambda shp: pl.BlockSpec(shp, lambda i: (0,) * len(shp))
    rowblk = pl.BlockSpec((1, HID, RA), lambda i: (i, 0, 0))
    tokblk = pl.BlockSpec((RA, CPP), lambda i: (i, 0))

    tok1 = _patchify(img)
    tok2 = _patchify(aug_img.astype(jnp.bfloat16))

    nz1, z1, z2, idx3, gap3, mse = pl.pallas_call(
        _kernel_a,
        grid=(GA,),
        in_specs=[
            tokblk, tokblk,
            full((CPP, FEAT)),
            full((FEAT, FEAT)), full((FEAT, HID)), full((FEAT, HID)),
            full((FEAT, FEAT)), full((FEAT, HID)), full((FEAT, HID)),
            full((K, HID)),
        ],
        out_specs=[
            rowblk, rowblk, rowblk,
            pl.BlockSpec((1, 1, RA), lambda i: (i, 0, 0)),
            pl.BlockSpec((1, 1, RA), lambda i: (i, 0, 0)),
            pl.BlockSpec((1, 1), lambda i: (0, 0)),
        ],
        out_shape=[
            jax.ShapeDtypeStruct((GA, HID, RA), jnp.float32),
            jax.ShapeDtypeStruct((GA, HID, RA), jnp.float32),
            jax.ShapeDtypeStruct((GA, HID, RA), jnp.float32),
            jax.ShapeDtypeStruct((GA, 1, RA), jnp.int32),
            jax.ShapeDtypeStruct((GA, 1, RA), jnp.float32),
            jax.ShapeDtypeStruct((1, 1), jnp.float32),
        ],
    )(tok1, tok2, Wp, W1, W2, Ws, W1e, W2e, Wse, centroid)

    qflat = queue.reshape(NQ, HID)
    nce = pl.pallas_call(
        _kernel_b,
        grid=(GB,),
        in_specs=[
            pl.BlockSpec((RB, HID), lambda i: (i, 0)),
            full((K, HID)),
        ],
        out_specs=pl.BlockSpec((1, 1), lambda i: (0, 0)),
        out_shape=jax.ShapeDtypeStruct((1, 1), jnp.float32),
    )(qflat, centroid)

    out = nz1.reshape(B, HID, HP, HP)
    z1o = z1.reshape(B, HID, HP, HP)
    z2o = z2.reshape(B, HID, HP, HP)
    return (out, z1o, z2o, mse[0, 0], nce[0, 0],
            idx3.reshape(N_TOK), gap3.reshape(N_TOK))


# speedup vs baseline: 3.0126x; 3.0126x over previous
import functools

import jax
import jax.numpy as jnp
from jax.experimental import pallas as pl
from jax.experimental.pallas import tpu as pltpu

B, C, HW, P = 16, 3, 384, 16
HP = HW // P
T = HP * HP
FEAT, HID = 384, 64
K, NS = 512, 100
MOM, TS = 0.99, 0.07

N_TOK = B * T
CPP = C * P * P
RA = T
GA = B
NQ = K * NS
RB = 2048
GB = NQ // RB

_NEG_BIG = -3.0e38


def _norm_rows(x):
    n = jnp.sqrt(jnp.sum(x * x, axis=-1, keepdims=True))
    return x / jnp.clip(n, 1e-12)


def _patchify(x):
    x = x.reshape(B, C, HP, P, HP, P).transpose(0, 2, 4, 1, 3, 5)
    return x.reshape(B * T, CPP)


def _kernel_a(tok1_ref, tok2_ref,
              wp_ref, w1_ref, w2_ref, ws_ref,
              w1e_ref, w2e_ref, wse_ref, cent_ref,
              nz1_ref, z1_ref, z2_ref, idx_ref, gap_ref, mse_ref):
    i = pl.program_id(0)
    f32 = jnp.float32
    t1 = tok1_ref[...]
    t2 = tok2_ref[...]

    x1 = jnp.dot(t1, wp_ref[...], preferred_element_type=f32)
    h1 = jnp.dot(jnp.maximum(jnp.dot(x1, w1_ref[...], preferred_element_type=f32), 0.0),
                 w2_ref[...], preferred_element_type=f32)
    h1 = h1 + jnp.dot(x1, ws_ref[...], preferred_element_type=f32)
    z1_ref[0] = h1.T
    nz1 = _norm_rows(h1)
    nz1_ref[0] = nz1.T

    w1n = MOM * w1e_ref[...] + (1.0 - MOM) * w1_ref[...]
    w2n = MOM * w2e_ref[...] + (1.0 - MOM) * w2_ref[...]
    wsn = MOM * wse_ref[...] + (1.0 - MOM) * ws_ref[...]
    x2 = jnp.dot(t2, wp_ref[...], preferred_element_type=f32)
    h2 = jnp.dot(jnp.maximum(jnp.dot(x2, w1n, preferred_element_type=f32), 0.0),
                 w2n, preferred_element_type=f32)
    h2 = h2 + jnp.dot(x2, wsn, preferred_element_type=f32)
    z2_ref[0] = h2.T
    nz2 = _norm_rows(h2)

    d = nz1 - nz2
    mse_part = jnp.sum(d * d) * (1.0 / (N_TOK * HID))

    cn = _norm_rows(cent_ref[...])
    cn2 = jnp.sum(cn * cn, axis=1)
    rn2 = jnp.sum(nz1 * nz1, axis=1, keepdims=True)
    s = jax.lax.dot_general(nz1, cn, (((1,), (1,)), ((), ())),
                            preferred_element_type=f32)
    neg = 2.0 * s - rn2 - cn2[None, :]
    m1 = jnp.max(neg, axis=1, keepdims=True)
    col = jax.lax.broadcasted_iota(jnp.int32, (RA, K), 1)
    idxv = jnp.min(jnp.where(neg == m1, col, K), axis=1)
    neg2 = jnp.where(col == idxv[:, None], _NEG_BIG, neg)
    m2 = jnp.max(neg2, axis=1)
    idx_ref[0, 0, :] = idxv
    gap_ref[0, 0, :] = m1[:, 0] - m2

    @pl.when(i == 0)
    def _():
        mse_ref[...] = mse_part.reshape(1, 1)

    @pl.when(i > 0)
    def _():
        mse_ref[...] += mse_part.reshape(1, 1)


def _kernel_b(q_ref, cent_ref, nce_ref):
    i = pl.program_id(0)
    f32 = jnp.float32

    qn = _norm_rows(q_ref[...])
    cn = _norm_rows(cent_ref[...])
    logits = jax.lax.dot_general(qn, cn, (((1,), (1,)), ((), ())),
                                 preferred_element_type=f32) * (1.0 / TS)
    lse = jnp.log(jnp.sum(jnp.exp(logits), axis=1))
    rows = i * RB + jax.lax.broadcasted_iota(jnp.int32, (RB, 1), 0)
    col = jax.lax.broadcasted_iota(jnp.int32, (RB, K), 1)
    hit = (rows >= NS * col) & (rows < NS * (col + 1))
    lab_logit = jnp.sum(jnp.where(hit, logits, 0.0), axis=1)
    part = jnp.sum(lse - lab_logit) * (1.0 / NQ)

    @pl.when(i == 0)
    def _():
        nce_ref[...] = part.reshape(1, 1)

    @pl.when(i > 0)
    def _():
        nce_ref[...] += part.reshape(1, 1)


@functools.partial(jax.jit)
def kernel(img, aug_img, Wp, W1, W2, Ws, W1e, W2e, Wse, centroid, queue):
    full = lambda shp: pl.BlockSpec(shp, lambda i: (0,) * len(shp))
    rowblk = pl.BlockSpec((1, HID, RA), lambda i: (i, 0, 0))
    tokblk = pl.BlockSpec((RA, CPP), lambda i: (i, 0))

    tok1 = _patchify(img)
    tok2 = _patchify(aug_img)

    nz1, z1, z2, idx3, gap3, mse = pl.pallas_call(
        _kernel_a,
        grid=(GA,),
        in_specs=[
            tokblk, tokblk,
            full((CPP, FEAT)),
            full((FEAT, FEAT)), full((FEAT, HID)), full((FEAT, HID)),
            full((FEAT, FEAT)), full((FEAT, HID)), full((FEAT, HID)),
            full((K, HID)),
        ],
        out_specs=[
            rowblk, rowblk, rowblk,
            pl.BlockSpec((1, 1, RA), lambda i: (i, 0, 0)),
            pl.BlockSpec((1, 1, RA), lambda i: (i, 0, 0)),
            pl.BlockSpec((1, 1), lambda i: (0, 0)),
        ],
        out_shape=[
            jax.ShapeDtypeStruct((GA, HID, RA), jnp.float32),
            jax.ShapeDtypeStruct((GA, HID, RA), jnp.float32),
            jax.ShapeDtypeStruct((GA, HID, RA), jnp.float32),
            jax.ShapeDtypeStruct((GA, 1, RA), jnp.int32),
            jax.ShapeDtypeStruct((GA, 1, RA), jnp.float32),
            jax.ShapeDtypeStruct((1, 1), jnp.float32),
        ],
    )(tok1, tok2, Wp, W1, W2, Ws, W1e, W2e, Wse, centroid)

    qflat = queue.reshape(NQ, HID)
    nce = pl.pallas_call(
        _kernel_b,
        grid=(GB,),
        in_specs=[
            pl.BlockSpec((RB, HID), lambda i: (i, 0)),
            full((K, HID)),
        ],
        out_specs=pl.BlockSpec((1, 1), lambda i: (0, 0)),
        out_shape=jax.ShapeDtypeStruct((1, 1), jnp.float32),
    )(qflat, centroid)

    out = nz1.reshape(B, HID, HP, HP)
    z1o = z1.reshape(B, HID, HP, HP)
    z2o = z2.reshape(B, HID, HP, HP)
    return (out, z1o, z2o, mse[0, 0], nce[0, 0],
            idx3.reshape(N_TOK), gap3.reshape(N_TOK))


# speedup vs baseline: 3.9653x; 1.3162x over previous
import functools

import jax
import jax.numpy as jnp
from jax.experimental import pallas as pl
from jax.experimental.pallas import tpu as pltpu

B, C, HW, P = 16, 3, 384, 16
HP = HW // P
T = HP * HP
FEAT, HID = 384, 64
K, NS = 512, 100
MOM, TS = 0.99, 0.07

N_TOK = B * T
CPP = C * P * P
HPH = HP // 2
RA = HPH * HP
NQ = K * NS
RB = 2048
GB = NQ // RB

_NEG_BIG = -3.0e38


def _norm_rows(x):
    n = jnp.sqrt(jnp.sum(x * x, axis=-1, keepdims=True))
    return x / jnp.clip(n, 1e-12)


def _kernel_a(*refs):
    t1p = refs[0:48]
    t2p = refs[48:96]
    (wp_ref, w1_ref, w2_ref, ws_ref, w1e_ref, w2e_ref, wse_ref,
     cent_ref) = refs[96:104]
    (nz1_ref, z1_ref, z2_ref, idx_ref, gap_ref, mse_ref) = refs[104:110]
    tok1_scr, tok2_scr = refs[110:112]

    b = pl.program_id(0)
    h = pl.program_id(1)
    f32 = jnp.float32

    for c in range(C):
        for p in range(P):
            col = c * P * P + p * P
            k = c * P + p
            tok1_scr[:, col:col + P] = t1p[k][...].reshape(RA, P)
            tok2_scr[:, col:col + P] = t2p[k][...].reshape(RA, P)
    t1 = tok1_scr[...]
    t2 = tok2_scr[...]

    x1 = jnp.dot(t1, wp_ref[...], preferred_element_type=f32)
    h1 = jnp.dot(jnp.maximum(jnp.dot(x1, w1_ref[...], preferred_element_type=f32), 0.0),
                 w2_ref[...], preferred_element_type=f32)
    h1 = h1 + jnp.dot(x1, ws_ref[...], preferred_element_type=f32)
    z1_ref[...] = h1
    nz1 = _norm_rows(h1)
    nz1_ref[...] = nz1

    w1n = MOM * w1e_ref[...] + (1.0 - MOM) * w1_ref[...]
    w2n = MOM * w2e_ref[...] + (1.0 - MOM) * w2_ref[...]
    wsn = MOM * wse_ref[...] + (1.0 - MOM) * ws_ref[...]
    x2 = jnp.dot(t2, wp_ref[...], preferred_element_type=f32)
    h2 = jnp.dot(jnp.maximum(jnp.dot(x2, w1n, preferred_element_type=f32), 0.0),
                 w2n, preferred_element_type=f32)
    h2 = h2 + jnp.dot(x2, wsn, preferred_element_type=f32)
    z2_ref[...] = h2
    nz2 = _norm_rows(h2)

    d = nz1 - nz2
    mse_part = jnp.sum(d * d) * (1.0 / (N_TOK * HID))

    cn = _norm_rows(cent_ref[...])
    cn2 = jnp.sum(cn * cn, axis=1)
    rn2 = jnp.sum(nz1 * nz1, axis=1, keepdims=True)
    s = jax.lax.dot_general(nz1, cn, (((1,), (1,)), ((), ())),
                            preferred_element_type=f32)
    neg = 2.0 * s - rn2 - cn2[None, :]
    m1 = jnp.max(neg, axis=1, keepdims=True)
    col = jax.lax.broadcasted_iota(jnp.int32, (RA, K), 1)
    idxv = jnp.min(jnp.where(neg == m1, col, K), axis=1)
    neg2 = jnp.where(col == idxv[:, None], _NEG_BIG, neg)
    m2 = jnp.max(neg2, axis=1)
    idx_ref[0, 0, :] = idxv
    gap_ref[0, 0, :] = m1[:, 0] - m2

    @pl.when((b + h) == 0)
    def _():
        mse_ref[...] = mse_part.reshape(1, 1)

    @pl.when((b + h) > 0)
    def _():
        mse_ref[...] += mse_part.reshape(1, 1)


def _kernel_b(q_ref, cent_ref, nce_ref):
    i = pl.program_id(0)
    f32 = jnp.float32

    qn = _norm_rows(q_ref[...])
    cn = _norm_rows(cent_ref[...])
    logits = jax.lax.dot_general(qn, cn, (((1,), (1,)), ((), ())),
                                 preferred_element_type=f32) * (1.0 / TS)
    lse = jnp.log(jnp.sum(jnp.exp(logits), axis=1))
    rows = i * RB + jax.lax.broadcasted_iota(jnp.int32, (RB, 1), 0)
    col = jax.lax.broadcasted_iota(jnp.int32, (RB, K), 1)
    hit = (rows >= NS * col) & (rows < NS * (col + 1))
    lab_logit = jnp.sum(jnp.where(hit, logits, 0.0), axis=1)
    part = jnp.sum(lse - lab_logit) * (1.0 / NQ)

    @pl.when(i == 0)
    def _():
        nce_ref[...] = part.reshape(1, 1)

    @pl.when(i > 0)
    def _():
        nce_ref[...] += part.reshape(1, 1)


@functools.partial(jax.jit)
def kernel(img, aug_img, Wp, W1, W2, Ws, W1e, W2e, Wse, centroid, queue):
    full = lambda shp: pl.BlockSpec(shp, lambda b, h: (0,) * len(shp))
    rowblk = pl.BlockSpec((RA, HID), lambda b, h: (b * 2 + h, 0))

    img5 = img.reshape(B, C, HP, P, HP, P)
    aug5 = aug_img.reshape(B, C, HP, P, HP, P)

    def slab_spec(c, p):
        return pl.BlockSpec((None, None, HPH, None, HP, P),
                            lambda b, h, c=c, p=p: (b, c, h, p, 0, 0))

    slab_specs = [slab_spec(c, p) for c in range(C) for p in range(P)]

    nz1, z1, z2, idx3, gap3, mse = pl.pallas_call(
        _kernel_a,
        grid=(B, 2),
        in_specs=(slab_specs + slab_specs
                  + [full((CPP, FEAT)),
                     full((FEAT, FEAT)), full((FEAT, HID)), full((FEAT, HID)),
                     full((FEAT, FEAT)), full((FEAT, HID)), full((FEAT, HID)),
                     full((K, HID))]),
        out_specs=[
            rowblk, rowblk, rowblk,
            pl.BlockSpec((1, 1, RA), lambda b, h: (b * 2 + h, 0, 0)),
            pl.BlockSpec((1, 1, RA), lambda b, h: (b * 2 + h, 0, 0)),
            pl.BlockSpec((1, 1), lambda b, h: (0, 0)),
        ],
        out_shape=[
            jax.ShapeDtypeStruct((N_TOK, HID), jnp.float32),
            jax.ShapeDtypeStruct((N_TOK, HID), jnp.float32),
            jax.ShapeDtypeStruct((N_TOK, HID), jnp.float32),
            jax.ShapeDtypeStruct((B * 2, 1, RA), jnp.int32),
            jax.ShapeDtypeStruct((B * 2, 1, RA), jnp.float32),
            jax.ShapeDtypeStruct((1, 1), jnp.float32),
        ],
        scratch_shapes=[
            pltpu.VMEM((RA, CPP), jnp.float32),
            pltpu.VMEM((RA, CPP), jnp.float32),
        ],
    )(*([img5] * 48 + [aug5] * 48
        + [Wp, W1, W2, Ws, W1e, W2e, Wse, centroid]))

    qflat = queue.reshape(NQ, HID)
    nce = pl.pallas_call(
        _kernel_b,
        grid=(GB,),
        in_specs=[
            pl.BlockSpec((RB, HID), lambda i: (i, 0)),
            pl.BlockSpec((K, HID), lambda i: (0, 0)),
        ],
        out_specs=pl.BlockSpec((1, 1), lambda i: (0, 0)),
        out_shape=jax.ShapeDtypeStruct((1, 1), jnp.float32),
    )(qflat, centroid)

    out = nz1.reshape(B, HP, HP, HID).transpose(0, 3, 1, 2)
    z1o = z1.reshape(B, HP, HP, HID).transpose(0, 3, 1, 2)
    z2o = z2.reshape(B, HP, HP, HID).transpose(0, 3, 1, 2)
    return (out, z1o, z2o, mse[0, 0], nce[0, 0],
            idx3.reshape(N_TOK), gap3.reshape(N_TOK))
